# Initial kernel scaffold; baseline (speedup 1.0000x reference)
import jax
import jax.numpy as jnp
from jax.experimental import pallas as pl


def kernel(x, duration, max_len):
    raise NotImplementedError("write your pallas kernel here")


# speedup vs baseline: 6.9549x; 6.9549x over previous
import functools

import jax
import jax.numpy as jnp
from jax import lax
from jax.experimental import pallas as pl
from jax.experimental.pallas import tpu as pltpu
from jax.experimental.pallas import tpu_sc as plsc

MAX_MEL = 2000
B, S, H = 16, 512, 256
HALF = MAX_MEL // 2
NCHUNK = 8
CHUNK = 128
TAIL = HALF - (NCHUNK - 1) * CHUNK
PADW = 1024
NLANE = 16


def _lr_body(x_hbm, dur_hbm, out_hbm, mel_hbm,
             dur_v, cums_v, hist_v, idx_lin_v, idxbuf_v,
             rows0_v, rows1_v, mel_v, dall_v, sem0, sem1):
    c = lax.axis_index("c")
    s = lax.axis_index("s")
    w = c * 16 + s
    b = w >> 1
    half = w & 1
    mlo = half * HALF

    pltpu.sync_copy(dur_hbm.at[pl.ds(b * S, S)], dur_v)
    carry = jnp.int32(0)
    base = jnp.int32(0)
    for i in range(S // NLANE):
        v = dur_v[pl.ds(i * NLANE, NLANE)]
        cv = plsc.cumsum(v) + carry
        cums_v[pl.ds(i * NLANE, NLANE)] = cv
        carry = carry + jnp.sum(v)
        base = base + jnp.sum((cv < mlo).astype(jnp.int32))
    cums_v[pl.ds(S, NLANE)] = jnp.full((NLANE,), -1, jnp.int32)

    zeros = jnp.zeros((NLANE,), jnp.int32)
    for j in range(PADW // NLANE):
        hist_v[pl.ds(j * NLANE, NLANE)] = zeros
    lane = lax.iota(jnp.int32, NLANE)
    for i in range(S // NLANE):
        cur = cums_v[pl.ds(i * NLANE, NLANE)]
        nxt = cums_v[pl.ds(i * NLANE + 1, NLANE)]
        pos = cur - mlo
        msk = (nxt != cur) & (pos >= 0) & (pos < PADW)
        plsc.store_scatter(hist_v, [pos], lane + (i * NLANE + 1), mask=msk)

    run = base
    rowbase = b * S
    for j in range(PADW // NLANE):
        v = hist_v[pl.ds(j * NLANE, NLANE)]
        cm = jnp.maximum(plsc.cummax(v), run)
        run = jnp.max(cm)
        idx_lin_v[pl.ds(j * NLANE, NLANE)] = jnp.minimum(cm, S - 1) + rowbase

    for j in range(NCHUNK):
        for k in range(CHUNK // NLANE):
            idxbuf_v[j, pl.ds(k * NLANE, NLANE)] = (
                idx_lin_v[pl.ds(j * CHUNK + k * NLANE, NLANE)])

    @pl.when((w & 15) == 0)
    def _mel():
        gb = (w >> 4) * 8
        pltpu.sync_copy(dur_hbm.at[pl.ds(gb * S, 8 * S)], dall_v)
        mel_vec = jnp.zeros((NLANE,), jnp.int32)
        for bb in range(8):
            acc = jnp.zeros((NLANE,), jnp.int32)
            for i in range(S // NLANE):
                acc = acc + dall_v[pl.ds(bb * S + i * NLANE, NLANE)]
            t = jnp.minimum(jnp.sum(acc), MAX_MEL)
            mel_vec = jnp.where(lane == bb, t, mel_vec)
        mel_v[...] = mel_vec
        pltpu.sync_copy(mel_v.at[pl.ds(0, 8)], mel_hbm.at[pl.ds(gb, 8)])

    bufs = (rows0_v, rows1_v)
    sems = (sem0, sem1)
    gout = b * MAX_MEL + mlo
    handles = [None] * NCHUNK
    handles[0] = pltpu.async_copy(x_hbm.at[idxbuf_v.at[0]], rows0_v, sem0)
    for j in range(NCHUNK):
        if j + 1 < NCHUNK:
            handles[j + 1] = pltpu.async_copy(
                x_hbm.at[idxbuf_v.at[j + 1]], bufs[(j + 1) % 2],
                sems[(j + 1) % 2])
        handles[j].wait()
        nrows = CHUNK if j + 1 < NCHUNK else TAIL
        pltpu.sync_copy(bufs[j % 2].at[pl.ds(0, nrows)],
                        out_hbm.at[pl.ds(gout + j * CHUNK, nrows)])


@functools.partial(
    pl.kernel,
    out_type=(jax.ShapeDtypeStruct((B * MAX_MEL, H), jnp.float32),
              jax.ShapeDtypeStruct((B,), jnp.int32)),
    mesh=plsc.VectorSubcoreMesh(core_axis_name="c", subcore_axis_name="s"),
    scratch_types=(
        pltpu.VMEM((S,), jnp.int32),
        pltpu.VMEM((S + NLANE,), jnp.int32),
        pltpu.VMEM((PADW,), jnp.int32),
        pltpu.VMEM((PADW,), jnp.int32),
        pltpu.VMEM((NCHUNK, CHUNK), jnp.int32),
        pltpu.VMEM((CHUNK, H), jnp.float32),
        pltpu.VMEM((CHUNK, H), jnp.float32),
        pltpu.VMEM((NLANE,), jnp.int32),
        pltpu.VMEM((8 * S,), jnp.int32),
        pltpu.SemaphoreType.DMA,
        pltpu.SemaphoreType.DMA,
    ),
    compiler_params=pltpu.CompilerParams(needs_layout_passes=False),
)
def _lr_kernel(x_hbm, dur_hbm, out_hbm, mel_hbm, *scratch):
    _lr_body(x_hbm, dur_hbm, out_hbm, mel_hbm, *scratch)


def kernel(x, duration, max_len):
    del max_len
    out_flat, mel_len = _lr_kernel(x.reshape(B * S, H), duration.reshape(B * S))
    return out_flat.reshape(B, MAX_MEL, H), mel_len


# speedup vs baseline: 7.2594x; 1.0438x over previous
import functools

import jax
import jax.numpy as jnp
from jax import lax
from jax.experimental import pallas as pl
from jax.experimental.pallas import tpu as pltpu
from jax.experimental.pallas import tpu_sc as plsc

MAX_MEL = 2000
B, S, H = 16, 512, 256
HALF = MAX_MEL // 2
NCHUNK = 8
CHUNK = 128
TAIL = HALF - (NCHUNK - 1) * CHUNK
PADW = 1024
NLANE = 16


def _lr_body(x_hbm, dur_hbm, out_hbm, mel_hbm,
             dur_v, cums_v, hist_v, idxbuf_v,
             rows0_v, rows1_v, rows2_v, mel_v, dall_v,
             gsem0, gsem1, gsem2, wsem0, wsem1, wsem2):
    c = lax.axis_index("c")
    s = lax.axis_index("s")
    w = c * 16 + s
    b = w >> 1
    half = w & 1
    mlo = half * HALF

    pltpu.sync_copy(dur_hbm.at[pl.ds(b * S, S)], dur_v)
    carry = jnp.int32(0)
    base = jnp.int32(0)
    for i in range(S // NLANE):
        v = dur_v[pl.ds(i * NLANE, NLANE)]
        cv = plsc.cumsum(v) + carry
        cums_v[pl.ds(i * NLANE, NLANE)] = cv
        carry = carry + jnp.sum(v)
        base = base + jnp.sum((cv < mlo).astype(jnp.int32))
    cums_v[pl.ds(S, NLANE)] = jnp.full((NLANE,), -1, jnp.int32)

    zeros = jnp.zeros((NLANE,), jnp.int32)
    for j in range(PADW // NLANE):
        hist_v[pl.ds(j * NLANE, NLANE)] = zeros
    lane = lax.iota(jnp.int32, NLANE)
    for i in range(S // NLANE):
        cur = cums_v[pl.ds(i * NLANE, NLANE)]
        nxt = cums_v[pl.ds(i * NLANE + 1, NLANE)]
        pos = cur - mlo
        msk = (nxt != cur) & (pos >= 0) & (pos < PADW)
        plsc.store_scatter(hist_v, [pos], lane + (i * NLANE + 1), mask=msk)

    run = base
    rowbase = b * S
    for j in range(PADW // NLANE):
        v = hist_v[pl.ds(j * NLANE, NLANE)]
        cm = jnp.maximum(plsc.cummax(v), run)
        run = jnp.max(cm)
        idxbuf_v[j // (CHUNK // NLANE),
                 pl.ds((j % (CHUNK // NLANE)) * NLANE, NLANE)] = (
            jnp.minimum(cm, S - 1) + rowbase)

    @pl.when((w & 15) == 0)
    def _mel():
        gb = (w >> 4) * 8
        pltpu.sync_copy(dur_hbm.at[pl.ds(gb * S, 8 * S)], dall_v)
        mel_vec = jnp.zeros((NLANE,), jnp.int32)
        for bb in range(8):
            acc = jnp.zeros((NLANE,), jnp.int32)
            for i in range(S // NLANE):
                acc = acc + dall_v[pl.ds(bb * S + i * NLANE, NLANE)]
            t = jnp.minimum(jnp.sum(acc), MAX_MEL)
            mel_vec = jnp.where(lane == bb, t, mel_vec)
        mel_v[...] = mel_vec
        pltpu.sync_copy(mel_v.at[pl.ds(0, 8)], mel_hbm.at[pl.ds(gb, 8)])

    bufs = (rows0_v, rows1_v, rows2_v)
    gsems = (gsem0, gsem1, gsem2)
    wsems = (wsem0, wsem1, wsem2)
    gout = b * MAX_MEL + mlo
    gh = [None] * NCHUNK
    wh = [None] * NCHUNK

    def start_gather(j):
        nrows = CHUNK if j + 1 < NCHUNK else TAIL
        gh[j] = pltpu.async_copy(
            x_hbm.at[idxbuf_v.at[j, pl.ds(0, nrows)]],
            bufs[j % 3].at[pl.ds(0, nrows)], gsems[j % 3])

    start_gather(0)
    start_gather(1)
    for j in range(NCHUNK):
        if j + 2 < NCHUNK:
            if j >= 1:
                wh[j - 1].wait()
            start_gather(j + 2)
        gh[j].wait()
        nrows = CHUNK if j + 1 < NCHUNK else TAIL
        wh[j] = pltpu.async_copy(bufs[j % 3].at[pl.ds(0, nrows)],
                                 out_hbm.at[pl.ds(gout + j * CHUNK, nrows)],
                                 wsems[j % 3])
    wh[NCHUNK - 3].wait()
    wh[NCHUNK - 2].wait()
    wh[NCHUNK - 1].wait()


@functools.partial(
    pl.kernel,
    out_type=(jax.ShapeDtypeStruct((B * MAX_MEL, H), jnp.float32),
              jax.ShapeDtypeStruct((B,), jnp.int32)),
    mesh=plsc.VectorSubcoreMesh(core_axis_name="c", subcore_axis_name="s"),
    scratch_types=(
        pltpu.VMEM((S,), jnp.int32),
        pltpu.VMEM((S + NLANE,), jnp.int32),
        pltpu.VMEM((PADW,), jnp.int32),
        pltpu.VMEM((NCHUNK, CHUNK), jnp.int32),
        pltpu.VMEM((CHUNK, H), jnp.float32),
        pltpu.VMEM((CHUNK, H), jnp.float32),
        pltpu.VMEM((CHUNK, H), jnp.float32),
        pltpu.VMEM((NLANE,), jnp.int32),
        pltpu.VMEM((8 * S,), jnp.int32),
        pltpu.SemaphoreType.DMA,
        pltpu.SemaphoreType.DMA,
        pltpu.SemaphoreType.DMA,
        pltpu.SemaphoreType.DMA,
        pltpu.SemaphoreType.DMA,
        pltpu.SemaphoreType.DMA,
    ),
    compiler_params=pltpu.CompilerParams(needs_layout_passes=False),
)
def _lr_kernel(x_hbm, dur_hbm, out_hbm, mel_hbm, *scratch):
    _lr_body(x_hbm, dur_hbm, out_hbm, mel_hbm, *scratch)


def kernel(x, duration, max_len):
    del max_len
    out_flat, mel_len = _lr_kernel(x.reshape(B * S, H), duration.reshape(B * S))
    return out_flat.reshape(B, MAX_MEL, H), mel_len
